# baseline (speedup 1.0000x reference)
import functools

import jax
import jax.numpy as jnp
from jax import lax
from jax.experimental import pallas as pl
from jax.experimental.pallas import tpu as pltpu
from jax.experimental.pallas import tpu_sc as plsc

T_ROWS = 2000
DIM = 128
B = 16384

_info = plsc.get_sparse_core_info()
NC, NS, L = _info.num_cores, _info.num_subcores, _info.num_lanes
NW = NC * NS
B_PER_W = B // NW
CHUNK = 128
NCHUNK = B_PER_W // CHUNK


def _make_kernel():
    mesh = plsc.VectorSubcoreMesh(core_axis_name="c", subcore_axis_name="s")

    @functools.partial(
        pl.kernel,
        mesh=mesh,
        out_type=jax.ShapeDtypeStruct((B, DIM), jnp.float32),
        scratch_types=[
            pltpu.VMEM((NCHUNK, CHUNK), jnp.int32),
            pltpu.VMEM((B_PER_W, DIM), jnp.float32),
            pltpu.SemaphoreType.DMA,
            pltpu.SemaphoreType.DMA,
        ],
    )
    def gather_kernel(t_hbm, table_hbm, out_hbm, idx_v, rows_v, gsem, ssem):
        wid = lax.axis_index("s") * NC + lax.axis_index("c")
        base = wid * B_PER_W
        pltpu.sync_copy(t_hbm.at[wid], idx_v)
        gathers = []
        for j in range(NCHUNK):
            gathers.append(
                pltpu.async_copy(
                    table_hbm.at[idx_v.at[j]],
                    rows_v.at[pl.ds(j * CHUNK, CHUNK)],
                    gsem,
                )
            )
        stores = []
        for j in range(NCHUNK):
            gathers[j].wait()
            stores.append(
                pltpu.async_copy(
                    rows_v.at[pl.ds(j * CHUNK, CHUNK)],
                    out_hbm.at[pl.ds(base + j * CHUNK, CHUNK)],
                    ssem,
                )
            )
        for d in stores:
            d.wait()

    return gather_kernel


_gather = _make_kernel()


@jax.jit
def kernel(t, pos_embeds):
    t_grouped = t.astype(jnp.int32).reshape(NW, NCHUNK, CHUNK)
    return _gather(t_grouped, pos_embeds)


# speedup vs baseline: 1.1179x; 1.1179x over previous
import functools

import jax
import jax.numpy as jnp
from jax import lax
from jax.experimental import pallas as pl
from jax.experimental.pallas import tpu as pltpu
from jax.experimental.pallas import tpu_sc as plsc

T_ROWS = 2000
DIM = 128
B = 16384

_info = plsc.get_sparse_core_info()
NC, NS, L = _info.num_cores, _info.num_subcores, _info.num_lanes
NW = NC * NS
B_PER_W = B // NW
CHUNK = 128
NCHUNK = B_PER_W // CHUNK
ROWS_PER_SUB = 128
LAST_ROWS = T_ROWS - (NS - 1) * ROWS_PER_SUB


def _make_kernel():
    mesh = plsc.VectorSubcoreMesh(core_axis_name="c", subcore_axis_name="s")

    @functools.partial(
        pl.kernel,
        mesh=mesh,
        out_type=jax.ShapeDtypeStruct((B, DIM), jnp.float32),
        scratch_types=[
            pltpu.VMEM((NCHUNK, CHUNK), jnp.int32),
            pltpu.VMEM((B_PER_W, DIM), jnp.float32),
            pltpu.VMEM_SHARED((T_ROWS, DIM), jnp.float32),
            pltpu.SemaphoreType.DMA,
            pltpu.SemaphoreType.DMA,
        ],
    )
    def gather_kernel(t_hbm, table_hbm, out_hbm, idx_v, rows_v, tab_s, gsem, ssem):
        cid = lax.axis_index("c")
        sid = lax.axis_index("s")
        wid = sid * NC + cid
        base = wid * B_PER_W
        tab_base = sid * ROWS_PER_SUB

        @pl.when(sid < NS - 1)
        def _stage_full():
            pltpu.sync_copy(
                table_hbm.at[pl.ds(tab_base, ROWS_PER_SUB)],
                tab_s.at[pl.ds(tab_base, ROWS_PER_SUB)],
            )

        @pl.when(sid == NS - 1)
        def _stage_tail():
            pltpu.sync_copy(
                table_hbm.at[pl.ds((NS - 1) * ROWS_PER_SUB, LAST_ROWS)],
                tab_s.at[pl.ds((NS - 1) * ROWS_PER_SUB, LAST_ROWS)],
            )

        pltpu.sync_copy(t_hbm.at[wid], idx_v)
        plsc.subcore_barrier()
        gathers = []
        for j in range(NCHUNK):
            gathers.append(
                pltpu.async_copy(
                    tab_s.at[idx_v.at[j]],
                    rows_v.at[pl.ds(j * CHUNK, CHUNK)],
                    gsem,
                )
            )
        stores = []
        for j in range(NCHUNK):
            gathers[j].wait()
            stores.append(
                pltpu.async_copy(
                    rows_v.at[pl.ds(j * CHUNK, CHUNK)],
                    out_hbm.at[pl.ds(base + j * CHUNK, CHUNK)],
                    ssem,
                )
            )
        for d in stores:
            d.wait()

    return gather_kernel


_gather = _make_kernel()


@jax.jit
def kernel(t, pos_embeds):
    t_grouped = t.astype(jnp.int32).reshape(NW, NCHUNK, CHUNK)
    return _gather(t_grouped, pos_embeds)


# speedup vs baseline: 1.1691x; 1.0458x over previous
import functools

import jax
import jax.numpy as jnp
from jax import lax
from jax.experimental import pallas as pl
from jax.experimental.pallas import tpu as pltpu
from jax.experimental.pallas import tpu_sc as plsc

T_ROWS = 2000
DIM = 128
B = 16384

_info = plsc.get_sparse_core_info()
NC, NS, L = _info.num_cores, _info.num_subcores, _info.num_lanes
NW = NC * NS
B_PER_W = B // NW
CHUNK = 128
NCHUNK = B_PER_W // CHUNK
ROWS_PER_SUB = 128


def _make_kernel():
    mesh = plsc.VectorSubcoreMesh(core_axis_name="c", subcore_axis_name="s")

    @functools.partial(
        pl.kernel,
        mesh=mesh,
        out_type=jax.ShapeDtypeStruct((B, DIM), jnp.float32),
        scratch_types=[
            pltpu.VMEM((NCHUNK, CHUNK), jnp.int32),
            pltpu.VMEM((B_PER_W, DIM), jnp.float32),
            pltpu.VMEM_SHARED((T_ROWS, DIM), jnp.float32),
            pltpu.SemaphoreType.DMA,
            pltpu.SemaphoreType.DMA,
            pltpu.SemaphoreType.DMA,
        ],
    )
    def gather_kernel(
        t_hbm, table_hbm, out_hbm, idx_v, rows_v, tab_s, gsem, ssem, tsem
    ):
        cid = lax.axis_index("c")
        sid = lax.axis_index("s")
        wid = sid * NC + cid
        base = wid * B_PER_W
        tab_base = jnp.minimum(sid * ROWS_PER_SUB, T_ROWS - ROWS_PER_SUB)
        stage = pltpu.async_copy(
            table_hbm.at[pl.ds(tab_base, ROWS_PER_SUB)],
            tab_s.at[pl.ds(tab_base, ROWS_PER_SUB)],
            tsem,
        )

        pltpu.sync_copy(t_hbm.at[wid], idx_v)
        gathers = [
            pltpu.async_copy(
                table_hbm.at[idx_v.at[0]], rows_v.at[pl.ds(0, CHUNK)], gsem
            )
        ]
        stage.wait()
        plsc.subcore_barrier()
        for j in range(1, NCHUNK):
            gathers.append(
                pltpu.async_copy(
                    tab_s.at[idx_v.at[j]],
                    rows_v.at[pl.ds(j * CHUNK, CHUNK)],
                    gsem,
                )
            )
        stores = []
        for j in range(NCHUNK):
            gathers[j].wait()
            stores.append(
                pltpu.async_copy(
                    rows_v.at[pl.ds(j * CHUNK, CHUNK)],
                    out_hbm.at[pl.ds(base + j * CHUNK, CHUNK)],
                    ssem,
                )
            )
        for d in stores:
            d.wait()

    return gather_kernel


_gather = _make_kernel()


@jax.jit
def kernel(t, pos_embeds):
    t_grouped = t.astype(jnp.int32).reshape(NW, NCHUNK, CHUNK)
    return _gather(t_grouped, pos_embeds)
